# baseline (speedup 1.0000x reference)
import functools

import jax
import jax.numpy as jnp
from jax import lax
from jax.experimental import pallas as pl
from jax.experimental.pallas import tpu as pltpu
from jax.experimental.pallas import tpu_sc as plsc

N = 50000
E = 1600000
G = 16
NPAD = 51200
RB = NPAD // 128
EPAD = 1638400
ROWS = EPAD // 128
ROWS_PER_TILE = ROWS // 32
CHUNKS = ROWS_PER_TILE // 16
USIZE = 852992

_mesh = plsc.VectorSubcoreMesh(
    core_axis_name="c", subcore_axis_name="s", num_cores=2, num_subcores=16)


def _sc_deg_body(dst_hbm, zn_hbm, out_hbm, dstb, ones, deg_sh):
    cid = lax.axis_index("c")
    sid = lax.axis_index("s")
    wid = cid * 16 + sid

    @pl.when(sid == 0)
    def _():
        pltpu.sync_copy(zn_hbm, deg_sh)

    for i in range(8):
        ones[pl.ds(i * 16, 16)] = jnp.full((16,), 1.0, jnp.float32)
    plsc.subcore_barrier()

    base = wid * ROWS_PER_TILE

    def chunk(c, carry):
        pltpu.sync_copy(dst_hbm.at[pl.ds(base + c * 16, 16)], dstb)

        def inner(j, carry2):
            pltpu.sync_copy(ones, deg_sh.at[dstb.at[j]], add=True)
            return carry2

        return lax.fori_loop(0, 16, inner, carry)

    lax.fori_loop(0, CHUNKS, chunk, 0)
    plsc.subcore_barrier()

    @pl.when(sid == 1)
    def _():
        pltpu.sync_copy(deg_sh, out_hbm.at[pl.ds(cid * NPAD, NPAD)])


_sc_deg = functools.partial(
    pl.kernel,
    out_type=jax.ShapeDtypeStruct((2 * NPAD,), jnp.float32),
    mesh=_mesh,
    scratch_types=[
        pltpu.VMEM((16, 128), jnp.int32),
        pltpu.VMEM((128,), jnp.float32),
        pltpu.VMEM_SHARED((NPAD,), jnp.float32),
    ],
)(_sc_deg_body)


def _tc_prep_body(degp_ref, xt_ref, bp_ref, yt_ref, dis_ref, didx_ref):
    deg = degp_ref[0] + degp_ref[1] + 1.0
    r = (lax.broadcasted_iota(jnp.int32, (RB, 128), 0) * 128
         + lax.broadcasted_iota(jnp.int32, (RB, 128), 1))
    dis = jnp.where(r < N, lax.rsqrt(deg), 0.0)
    dis_ref[...] = dis
    didx_ref[...] = bp_ref[...] * N
    yt_ref[...] = xt_ref[...] * dis[None]


_tc_prep = pl.pallas_call(
    _tc_prep_body,
    out_shape=[
        jax.ShapeDtypeStruct((3, RB, 128), jnp.float32),
        jax.ShapeDtypeStruct((RB, 128), jnp.float32),
        jax.ShapeDtypeStruct((RB, 128), jnp.int32),
    ],
)


def _sc_main_body(src_hbm, dst_hbm, y0_hbm, y1_hbm, y2_hbm, dis_hbm, didx_hbm,
                  zn_hbm, zu_hbm, t_out, u_out,
                  srcb, dstb, y0b, y1b, y2b, dvalbuf, didxbuf, uidx,
                  s0, s1, s2, s3, s4,
                  t0_sh, t1_sh, t2_sh, u_sh, y0_sh, y1_sh, y2_sh,
                  dis_sh, didx_sh):
    cid = lax.axis_index("c")
    sid = lax.axis_index("s")
    wid = cid * 16 + sid

    for k, (src_arr, dst_arr) in enumerate([
            (y0_hbm, y0_sh), (y1_hbm, y1_sh), (y2_hbm, y2_sh),
            (dis_hbm, dis_sh), (didx_hbm, didx_sh),
            (zn_hbm, t0_sh), (zn_hbm, t1_sh), (zn_hbm, t2_sh),
            (zu_hbm, u_sh)]):
        @pl.when(sid == k)
        def _(src_arr=src_arr, dst_arr=dst_arr):
            pltpu.sync_copy(src_arr, dst_arr)

    plsc.subcore_barrier()

    base = wid * ROWS_PER_TILE

    def chunk(c, carry):
        pltpu.sync_copy(src_hbm.at[pl.ds(base + c * 16, 16)], srcb)
        pltpu.sync_copy(dst_hbm.at[pl.ds(base + c * 16, 16)], dstb)

        def inner(j, carry2):
            sj = srcb.at[j]
            dj = dstb.at[j]
            cy0 = pltpu.async_copy(y0_sh.at[sj], y0b, s0)
            cy1 = pltpu.async_copy(y1_sh.at[sj], y1b, s0)
            cy2 = pltpu.async_copy(y2_sh.at[sj], y2b, s0)
            cdv = pltpu.async_copy(dis_sh.at[dj], dvalbuf, s0)
            cdi = pltpu.async_copy(didx_sh.at[dj], didxbuf, s0)
            cdi.wait()
            for k in range(8):
                sl = pl.ds(k * 16, 16)
                uidx[0, sl] = didxbuf[sl] + srcb[j, sl]
            cy0.wait()
            cy1.wait()
            cy2.wait()
            cdv.wait()
            w0 = pltpu.async_copy(y0b, t0_sh.at[dj], s1, add=True)
            w1 = pltpu.async_copy(y1b, t1_sh.at[dj], s2, add=True)
            w2 = pltpu.async_copy(y2b, t2_sh.at[dj], s3, add=True)
            pltpu.sync_copy(dvalbuf, u_sh.at[uidx.at[0]], add=True)
            w0.wait()
            w1.wait()
            w2.wait()
            return carry2

        return lax.fori_loop(0, 16, inner, carry)

    lax.fori_loop(0, CHUNKS, chunk, 0)
    plsc.subcore_barrier()

    for k, plane in enumerate([t0_sh, t1_sh, t2_sh]):
        @pl.when(sid == 9 + k)
        def _(plane=plane, k=k):
            pltpu.sync_copy(plane,
                            t_out.at[pl.ds(cid * 3 * NPAD + k * NPAD, NPAD)])

    @pl.when(sid == 12)
    def _():
        pltpu.sync_copy(u_sh, u_out.at[pl.ds(cid * USIZE, USIZE)])


_sc_main = functools.partial(
    pl.kernel,
    out_type=(
        jax.ShapeDtypeStruct((2 * 3 * NPAD,), jnp.float32),
        jax.ShapeDtypeStruct((2 * USIZE,), jnp.float32),
    ),
    mesh=_mesh,
    scratch_types=[
        pltpu.VMEM((16, 128), jnp.int32),
        pltpu.VMEM((16, 128), jnp.int32),
        pltpu.VMEM((128,), jnp.float32),
        pltpu.VMEM((128,), jnp.float32),
        pltpu.VMEM((128,), jnp.float32),
        pltpu.VMEM((128,), jnp.float32),
        pltpu.VMEM((128,), jnp.int32),
        pltpu.VMEM((1, 128), jnp.int32),
        pltpu.SemaphoreType.DMA,
        pltpu.SemaphoreType.DMA,
        pltpu.SemaphoreType.DMA,
        pltpu.SemaphoreType.DMA,
        pltpu.SemaphoreType.DMA,
        pltpu.VMEM_SHARED((NPAD,), jnp.float32),
        pltpu.VMEM_SHARED((NPAD,), jnp.float32),
        pltpu.VMEM_SHARED((NPAD,), jnp.float32),
        pltpu.VMEM_SHARED((USIZE,), jnp.float32),
        pltpu.VMEM_SHARED((NPAD,), jnp.float32),
        pltpu.VMEM_SHARED((NPAD,), jnp.float32),
        pltpu.VMEM_SHARED((NPAD,), jnp.float32),
        pltpu.VMEM_SHARED((NPAD,), jnp.float32),
        pltpu.VMEM_SHARED((NPAD,), jnp.int32),
    ],
)(_sc_main_body)


NB = 6400
NBLK = NPAD // NB


def _tc_finish_body(tp_ref, yc_ref, disr_ref, up_ref, b2d_ref,
                    w1t_ref, b1_ref, w2_ref, b2_ref, wfc_ref, bfc_ref, q_ref,
                    pooled_acc, counts_acc):
    i = pl.program_id(0)
    disr = disr_ref[...]
    t_tot = tp_ref[0] + tp_ref[1] + yc_ref[...]
    agg1 = t_tot * disr
    h1t = jnp.maximum(
        jnp.dot(w1t_ref[...], agg1, preferred_element_type=jnp.float32)
        + b1_ref[...], 0.0)
    gi = lax.broadcasted_iota(jnp.int32, (G, NB), 0)
    m = (b2d_ref[...] == gi).astype(jnp.float32)
    u = up_ref[0] + up_ref[1]
    w = disr * (u + disr * m)
    pooled = lax.dot_general(w, h1t, (((1,), (1,)), ((), ())),
                             preferred_element_type=jnp.float32)
    counts = jnp.sum(m, axis=1, keepdims=True)

    @pl.when(i == 0)
    def _():
        pooled_acc[...] = jnp.zeros_like(pooled_acc)
        counts_acc[...] = jnp.zeros_like(counts_acc)

    pooled_acc[...] += pooled
    counts_acc[...] += counts

    @pl.when(i == NBLK - 1)
    def _():
        ge = pooled_acc[...] / jnp.maximum(counts_acc[...], 1.0)
        wc = jnp.dot(w2_ref[...], wfc_ref[...],
                     preferred_element_type=jnp.float32)
        bc = jnp.dot(b2_ref[...], wfc_ref[...],
                     preferred_element_type=jnp.float32) + bfc_ref[...]
        q_ref[...] = jnp.dot(ge, wc, preferred_element_type=jnp.float32) + bc


_tc_finish = pl.pallas_call(
    _tc_finish_body,
    grid=(NBLK,),
    in_specs=[
        pl.BlockSpec((2, 3, NB), lambda i: (0, 0, i)),
        pl.BlockSpec((3, NB), lambda i: (0, i)),
        pl.BlockSpec((1, NB), lambda i: (0, i)),
        pl.BlockSpec((2, G, NB), lambda i: (0, 0, i)),
        pl.BlockSpec((1, NB), lambda i: (0, i)),
        pl.BlockSpec((32, 3), lambda i: (0, 0)),
        pl.BlockSpec((32, 1), lambda i: (0, 0)),
        pl.BlockSpec((32, 32), lambda i: (0, 0)),
        pl.BlockSpec((1, 32), lambda i: (0, 0)),
        pl.BlockSpec((32, 256), lambda i: (0, 0)),
        pl.BlockSpec((1, 256), lambda i: (0, 0)),
    ],
    out_specs=pl.BlockSpec((G, 256), lambda i: (0, 0)),
    out_shape=jax.ShapeDtypeStruct((G, 256), jnp.float32),
    scratch_shapes=[
        pltpu.VMEM((G, 32), jnp.float32),
        pltpu.VMEM((G, 1), jnp.float32),
    ],
)


def kernel(x, edge_index, batch, W1, b1, W2, b2, Wfc, bfc):
    src = edge_index[0].astype(jnp.int32)
    dst = edge_index[1].astype(jnp.int32)
    b32 = batch.astype(jnp.int32)
    padv = jnp.full((EPAD - E,), N, jnp.int32)
    src_p = jnp.concatenate([src, padv]).reshape(ROWS, 128)
    dst_p = jnp.concatenate([dst, padv]).reshape(ROWS, 128)

    deg_flat = _sc_deg(dst_p, jnp.zeros((NPAD,), jnp.float32))

    xt = jnp.pad(x, ((0, NPAD - N), (0, 0))).T.reshape(3, RB, 128)
    batch_p = jnp.pad(b32, (0, NPAD - N), constant_values=G)
    yt, dis2, didx2 = _tc_prep(deg_flat.reshape(2, RB, 128), xt,
                               batch_p.reshape(RB, 128))
    yf = yt.reshape(3, NPAD)
    dis = dis2.reshape(NPAD)
    didx = didx2.reshape(NPAD)

    t_part, u_part = _sc_main(
        src_p, dst_p, yf[0], yf[1], yf[2], dis, didx,
        jnp.zeros((NPAD,), jnp.float32),
        jnp.zeros((USIZE,), jnp.float32))

    up = u_part.reshape(2, USIZE)[:, :G * N].reshape(2, G, N)
    up = jnp.pad(up, ((0, 0), (0, 0), (0, NPAD - N)))
    q = _tc_finish(
        t_part.reshape(2, 3, NPAD), yf, dis.reshape(1, NPAD), up,
        batch_p.reshape(1, NPAD), W1.T, b1.reshape(32, 1),
        W2, b2.reshape(1, 32), Wfc, bfc.reshape(1, 256))
    return q


# speedup vs baseline: 1.0423x; 1.0423x over previous
import functools

import jax
import jax.numpy as jnp
from jax import lax
from jax.experimental import pallas as pl
from jax.experimental.pallas import tpu as pltpu
from jax.experimental.pallas import tpu_sc as plsc

N = 50000
E = 1600000
G = 16
NPAD = 51200
RB = NPAD // 128
EPAD = 1638400
ROWS = EPAD // 128
ROWS_PER_TILE = ROWS // 32
CHUNKS = ROWS_PER_TILE // 16
USIZE = 852992

_mesh = plsc.VectorSubcoreMesh(
    core_axis_name="c", subcore_axis_name="s", num_cores=2, num_subcores=16)


def _sc_deg_body(dst_hbm, zn_hbm, out_hbm, dstb, ones, deg_sh):
    cid = lax.axis_index("c")
    sid = lax.axis_index("s")
    wid = cid * 16 + sid

    @pl.when(sid == 0)
    def _():
        pltpu.sync_copy(zn_hbm, deg_sh)

    for i in range(8):
        ones[pl.ds(i * 16, 16)] = jnp.full((16,), 1.0, jnp.float32)
    plsc.subcore_barrier()

    base = wid * ROWS_PER_TILE

    def chunk(c, carry):
        pltpu.sync_copy(dst_hbm.at[pl.ds(base + c * 16, 16)], dstb)

        def inner(j, carry2):
            pltpu.sync_copy(ones, deg_sh.at[dstb.at[j]], add=True)
            return carry2

        return lax.fori_loop(0, 16, inner, carry)

    lax.fori_loop(0, CHUNKS, chunk, 0)
    plsc.subcore_barrier()

    @pl.when(sid == 1)
    def _():
        pltpu.sync_copy(deg_sh, out_hbm.at[pl.ds(cid * NPAD, NPAD)])


_sc_deg = functools.partial(
    pl.kernel,
    out_type=jax.ShapeDtypeStruct((2 * NPAD,), jnp.float32),
    mesh=_mesh,
    scratch_types=[
        pltpu.VMEM((16, 128), jnp.int32),
        pltpu.VMEM((128,), jnp.float32),
        pltpu.VMEM_SHARED((NPAD,), jnp.float32),
    ],
)(_sc_deg_body)


def _tc_prep_body(degp_ref, xt_ref, bp_ref, yt_ref, dis_ref, didx_ref):
    deg = degp_ref[0] + degp_ref[1] + 1.0
    r = (lax.broadcasted_iota(jnp.int32, (RB, 128), 0) * 128
         + lax.broadcasted_iota(jnp.int32, (RB, 128), 1))
    dis = jnp.where(r < N, lax.rsqrt(deg), 0.0)
    dis_ref[...] = dis
    didx_ref[...] = bp_ref[...] * N
    yt_ref[...] = xt_ref[...] * dis[None]


_tc_prep = pl.pallas_call(
    _tc_prep_body,
    out_shape=[
        jax.ShapeDtypeStruct((3, RB, 128), jnp.float32),
        jax.ShapeDtypeStruct((RB, 128), jnp.float32),
        jax.ShapeDtypeStruct((RB, 128), jnp.int32),
    ],
)


def _sc_main_body(src_hbm, dst_hbm, y0_hbm, y1_hbm, y2_hbm, dis_hbm, didx_hbm,
                  zn_hbm, zu_hbm, t_out, u_out,
                  srcb, dstb, y0b, y1b, y2b, dvalbuf, didxbuf, uidx,
                  s0, s1, s2, s3, s4,
                  t0_sh, t1_sh, t2_sh, u_sh, y0_sh, y1_sh, y2_sh,
                  dis_sh, didx_sh):
    cid = lax.axis_index("c")
    sid = lax.axis_index("s")
    wid = cid * 16 + sid

    for k, (src_arr, dst_arr) in enumerate([
            (y0_hbm, y0_sh), (y1_hbm, y1_sh), (y2_hbm, y2_sh),
            (dis_hbm, dis_sh), (didx_hbm, didx_sh),
            (zn_hbm, t0_sh), (zn_hbm, t1_sh), (zn_hbm, t2_sh),
            (zu_hbm, u_sh)]):
        @pl.when(sid == k)
        def _(src_arr=src_arr, dst_arr=dst_arr):
            pltpu.sync_copy(src_arr, dst_arr)

    plsc.subcore_barrier()

    base = wid * ROWS_PER_TILE

    def chunk(c, carry):
        pltpu.sync_copy(src_hbm.at[pl.ds(base + c * 16, 16)], srcb)
        pltpu.sync_copy(dst_hbm.at[pl.ds(base + c * 16, 16)], dstb)

        def inner(j, carry2):
            sj = srcb.at[j]
            dj = dstb.at[j]
            cy0 = pltpu.async_copy(y0_sh.at[sj], y0b, s0)
            cy1 = pltpu.async_copy(y1_sh.at[sj], y1b, s1)
            cy2 = pltpu.async_copy(y2_sh.at[sj], y2b, s2)
            cdv = pltpu.async_copy(dis_sh.at[dj], dvalbuf, s3)
            cdi = pltpu.async_copy(didx_sh.at[dj], didxbuf, s4)
            cy0.wait()
            pltpu.sync_copy(y0b, t0_sh.at[dj], add=True)
            cy1.wait()
            pltpu.sync_copy(y1b, t1_sh.at[dj], add=True)
            cy2.wait()
            pltpu.sync_copy(y2b, t2_sh.at[dj], add=True)
            cdi.wait()
            for k in range(8):
                sl = pl.ds(k * 16, 16)
                uidx[0, sl] = didxbuf[sl] + srcb[j, sl]
            cdv.wait()
            pltpu.sync_copy(dvalbuf, u_sh.at[uidx.at[0]], add=True)
            return carry2

        return lax.fori_loop(0, 16, inner, carry)

    lax.fori_loop(0, CHUNKS, chunk, 0)
    plsc.subcore_barrier()

    for k, plane in enumerate([t0_sh, t1_sh, t2_sh]):
        @pl.when(sid == 9 + k)
        def _(plane=plane, k=k):
            pltpu.sync_copy(plane,
                            t_out.at[pl.ds(cid * 3 * NPAD + k * NPAD, NPAD)])

    @pl.when(sid == 12)
    def _():
        pltpu.sync_copy(u_sh, u_out.at[pl.ds(cid * USIZE, USIZE)])


_sc_main = functools.partial(
    pl.kernel,
    out_type=(
        jax.ShapeDtypeStruct((2 * 3 * NPAD,), jnp.float32),
        jax.ShapeDtypeStruct((2 * USIZE,), jnp.float32),
    ),
    mesh=_mesh,
    scratch_types=[
        pltpu.VMEM((16, 128), jnp.int32),
        pltpu.VMEM((16, 128), jnp.int32),
        pltpu.VMEM((128,), jnp.float32),
        pltpu.VMEM((128,), jnp.float32),
        pltpu.VMEM((128,), jnp.float32),
        pltpu.VMEM((128,), jnp.float32),
        pltpu.VMEM((128,), jnp.int32),
        pltpu.VMEM((1, 128), jnp.int32),
        pltpu.SemaphoreType.DMA,
        pltpu.SemaphoreType.DMA,
        pltpu.SemaphoreType.DMA,
        pltpu.SemaphoreType.DMA,
        pltpu.SemaphoreType.DMA,
        pltpu.VMEM_SHARED((NPAD,), jnp.float32),
        pltpu.VMEM_SHARED((NPAD,), jnp.float32),
        pltpu.VMEM_SHARED((NPAD,), jnp.float32),
        pltpu.VMEM_SHARED((USIZE,), jnp.float32),
        pltpu.VMEM_SHARED((NPAD,), jnp.float32),
        pltpu.VMEM_SHARED((NPAD,), jnp.float32),
        pltpu.VMEM_SHARED((NPAD,), jnp.float32),
        pltpu.VMEM_SHARED((NPAD,), jnp.float32),
        pltpu.VMEM_SHARED((NPAD,), jnp.int32),
    ],
)(_sc_main_body)


NB = 6400
NBLK = NPAD // NB


def _tc_finish_body(tp_ref, yc_ref, disr_ref, up_ref, b2d_ref,
                    w1t_ref, b1_ref, w2_ref, b2_ref, wfc_ref, bfc_ref, q_ref,
                    pooled_acc, counts_acc):
    i = pl.program_id(0)
    disr = disr_ref[...]
    t_tot = tp_ref[0] + tp_ref[1] + yc_ref[...]
    agg1 = t_tot * disr
    h1t = jnp.maximum(
        jnp.dot(w1t_ref[...], agg1, preferred_element_type=jnp.float32)
        + b1_ref[...], 0.0)
    gi = lax.broadcasted_iota(jnp.int32, (G, NB), 0)
    m = (b2d_ref[...] == gi).astype(jnp.float32)
    u = up_ref[0] + up_ref[1]
    w = disr * (u + disr * m)
    pooled = lax.dot_general(w, h1t, (((1,), (1,)), ((), ())),
                             preferred_element_type=jnp.float32)
    counts = jnp.sum(m, axis=1, keepdims=True)

    @pl.when(i == 0)
    def _():
        pooled_acc[...] = jnp.zeros_like(pooled_acc)
        counts_acc[...] = jnp.zeros_like(counts_acc)

    pooled_acc[...] += pooled
    counts_acc[...] += counts

    @pl.when(i == NBLK - 1)
    def _():
        ge = pooled_acc[...] / jnp.maximum(counts_acc[...], 1.0)
        wc = jnp.dot(w2_ref[...], wfc_ref[...],
                     preferred_element_type=jnp.float32)
        bc = jnp.dot(b2_ref[...], wfc_ref[...],
                     preferred_element_type=jnp.float32) + bfc_ref[...]
        q_ref[...] = jnp.dot(ge, wc, preferred_element_type=jnp.float32) + bc


_tc_finish = pl.pallas_call(
    _tc_finish_body,
    grid=(NBLK,),
    in_specs=[
        pl.BlockSpec((2, 3, NB), lambda i: (0, 0, i)),
        pl.BlockSpec((3, NB), lambda i: (0, i)),
        pl.BlockSpec((1, NB), lambda i: (0, i)),
        pl.BlockSpec((2, G, NB), lambda i: (0, 0, i)),
        pl.BlockSpec((1, NB), lambda i: (0, i)),
        pl.BlockSpec((32, 3), lambda i: (0, 0)),
        pl.BlockSpec((32, 1), lambda i: (0, 0)),
        pl.BlockSpec((32, 32), lambda i: (0, 0)),
        pl.BlockSpec((1, 32), lambda i: (0, 0)),
        pl.BlockSpec((32, 256), lambda i: (0, 0)),
        pl.BlockSpec((1, 256), lambda i: (0, 0)),
    ],
    out_specs=pl.BlockSpec((G, 256), lambda i: (0, 0)),
    out_shape=jax.ShapeDtypeStruct((G, 256), jnp.float32),
    scratch_shapes=[
        pltpu.VMEM((G, 32), jnp.float32),
        pltpu.VMEM((G, 1), jnp.float32),
    ],
)


def kernel(x, edge_index, batch, W1, b1, W2, b2, Wfc, bfc):
    src = edge_index[0].astype(jnp.int32)
    dst = edge_index[1].astype(jnp.int32)
    b32 = batch.astype(jnp.int32)
    padv = jnp.full((EPAD - E,), N, jnp.int32)
    src_p = jnp.concatenate([src, padv]).reshape(ROWS, 128)
    dst_p = jnp.concatenate([dst, padv]).reshape(ROWS, 128)

    deg_flat = _sc_deg(dst_p, jnp.zeros((NPAD,), jnp.float32))

    xt = jnp.pad(x, ((0, NPAD - N), (0, 0))).T.reshape(3, RB, 128)
    batch_p = jnp.pad(b32, (0, NPAD - N), constant_values=G)
    yt, dis2, didx2 = _tc_prep(deg_flat.reshape(2, RB, 128), xt,
                               batch_p.reshape(RB, 128))
    yf = yt.reshape(3, NPAD)
    dis = dis2.reshape(NPAD)
    didx = didx2.reshape(NPAD)

    t_part, u_part = _sc_main(
        src_p, dst_p, yf[0], yf[1], yf[2],
        dis, didx,
        jnp.zeros((NPAD,), jnp.float32),
        jnp.zeros((USIZE,), jnp.float32))

    up = u_part.reshape(2, USIZE)[:, :G * N].reshape(2, G, N)
    up = jnp.pad(up, ((0, 0), (0, 0), (0, NPAD - N)))
    q = _tc_finish(
        t_part.reshape(2, 3, NPAD), yf, dis.reshape(1, NPAD), up,
        batch_p.reshape(1, NPAD), W1.T, b1.reshape(32, 1),
        W2, b2.reshape(1, 32), Wfc, bfc.reshape(1, 256))
    return q
